# initial kernel scaffold (unmeasured)
import jax
import jax.numpy as jnp
from jax import lax
from jax.experimental import pallas as pl
from jax.experimental.pallas import tpu as pltpu

N_DEV = 8
HEADS_PER = 8
DH = 128
SQ = 1024
BLK = 64
SCALE = 0.08838834764831843


def kernel(x, Wq, K_ext, V_ext, Wo):
    def body(x_ref, wq_ref, k_hbm, v_hbm, wo_ref, out_ref,
             comm_wq, comm_wo, kbuf, vbuf, ctx_ref,
             wq_send, wq_recv, wo_send, wo_recv, ksem, vsem):
        my = lax.axis_index("i")
        left = lax.rem(my + N_DEV - 1, N_DEV)
        right = lax.rem(my + 1, N_DEV)

        barrier = pltpu.get_barrier_semaphore()
        for nbr in (left, right):
            pl.semaphore_signal(
                barrier, inc=1,
                device_id=(nbr,), device_id_type=pl.DeviceIdType.MESH,
            )
        pl.semaphore_wait(barrier, 2)

        def kv_descs(t, slot):
            g = jnp.mod(my - t, N_DEV)
            descs = []
            for hh in range(HEADS_PER):
                descs.append(pltpu.make_async_copy(
                    k_hbm.at[my, :, g * HEADS_PER + hh, :],
                    kbuf.at[slot, hh], ksem.at[slot]))
                descs.append(pltpu.make_async_copy(
                    v_hbm.at[my, :, g * HEADS_PER + hh, :],
                    vbuf.at[slot, hh], vsem.at[slot]))
            return descs

        comm_wq[0] = wq_ref[...]
        comm_wo[0] = wo_ref[...]
        for d in kv_descs(0, 0):
            d.start()

        x2 = x_ref[0]

        row = lax.broadcasted_iota(jnp.int32, (SQ, SQ), 0) // BLK
        col = lax.broadcasted_iota(jnp.int32, (SQ, SQ), 1) // BLK
        maskadd = jnp.where(col <= row, 0.0, -1e9).astype(jnp.float32)

        out_ref[0] = jnp.zeros((SQ, SQ), jnp.float32)

        def step(t, carry):
            cur = lax.rem(t, 3)
            nxt = lax.rem(t + 1, 3)
            slot = lax.rem(t, 2)
            nslot = lax.rem(t + 1, 2)

            send_wq = pltpu.make_async_remote_copy(
                src_ref=comm_wq.at[cur], dst_ref=comm_wq.at[nxt],
                send_sem=wq_send.at[cur], recv_sem=wq_recv.at[nxt],
                device_id=(right,), device_id_type=pl.DeviceIdType.MESH)
            send_wo = pltpu.make_async_remote_copy(
                src_ref=comm_wo.at[cur], dst_ref=comm_wo.at[nxt],
                send_sem=wo_send.at[cur], recv_sem=wo_recv.at[nxt],
                device_id=(right,), device_id_type=pl.DeviceIdType.MESH)

            @pl.when(t < N_DEV - 1)
            def _():
                send_wq.start()
                send_wo.start()
                for d in kv_descs(t + 1, nslot):
                    d.start()

            for d in kv_descs(t, slot):
                d.wait()

            q = jnp.dot(x2, comm_wq[cur], preferred_element_type=jnp.float32)
            for hh in range(HEADS_PER):
                qh = q[:, hh * DH:(hh + 1) * DH]
                s = lax.dot_general(
                    qh, kbuf[slot, hh], (((1,), (1,)), ((), ())),
                    preferred_element_type=jnp.float32)
                s = s * SCALE + maskadd
                m = jnp.max(s, axis=-1, keepdims=True)
                w = jnp.exp(s - m)
                w = w * (1.0 / jnp.sum(w, axis=-1, keepdims=True))
                ctx_ref[:, hh * DH:(hh + 1) * DH] = jnp.dot(
                    w, vbuf[slot, hh], preferred_element_type=jnp.float32)
            out_ref[0] += jnp.dot(
                ctx_ref[...], comm_wo[cur], preferred_element_type=jnp.float32)

            @pl.when(t < N_DEV - 1)
            def _():
                send_wq.wait()
                send_wo.wait()

            return carry

        lax.fori_loop(0, N_DEV, step, 0)

    return pl.pallas_call(
        body,
        out_shape=jax.ShapeDtypeStruct((1, SQ, SQ), jnp.float32),
        in_specs=[
            pl.BlockSpec(memory_space=pltpu.VMEM),
            pl.BlockSpec(memory_space=pltpu.VMEM),
            pl.BlockSpec(memory_space=pltpu.ANY),
            pl.BlockSpec(memory_space=pltpu.ANY),
            pl.BlockSpec(memory_space=pltpu.VMEM),
        ],
        out_specs=pl.BlockSpec(memory_space=pltpu.VMEM),
        scratch_shapes=[
            pltpu.VMEM((3, SQ, SQ), jnp.float32),
            pltpu.VMEM((3, SQ, SQ), jnp.float32),
            pltpu.VMEM((2, HEADS_PER, SQ, DH), jnp.float32),
            pltpu.VMEM((2, HEADS_PER, SQ, DH), jnp.float32),
            pltpu.VMEM((SQ, HEADS_PER * DH), jnp.float32),
            pltpu.SemaphoreType.DMA((3,)),
            pltpu.SemaphoreType.DMA((3,)),
            pltpu.SemaphoreType.DMA((3,)),
            pltpu.SemaphoreType.DMA((3,)),
            pltpu.SemaphoreType.DMA((2,)),
            pltpu.SemaphoreType.DMA((2,)),
        ],
        compiler_params=pltpu.CompilerParams(collective_id=0),
    )(x, Wq, K_ext, V_ext, Wo)


# baseline (device time: 852270 ns/iter reference)
import jax
import jax.numpy as jnp
from jax import lax
from jax.experimental import pallas as pl
from jax.experimental.pallas import tpu as pltpu

N_DEV = 8
HEADS_PER = 8
DH = 128
SQ = 1024
BLK = 64
SCALE = 0.08838834764831843


def kernel(x, Wq, K_ext, V_ext, Wo):
    def body(x_ref, wq_ref, k_hbm, v_hbm, wo_ref, out_ref,
             comm_wq, comm_wo, kbuf, vbuf, ctx_ref,
             wq_send, wq_recv, wo_send, wo_recv, ksem, vsem):
        my = lax.axis_index("i")
        left = lax.rem(my + N_DEV - 1, N_DEV)
        right = lax.rem(my + 1, N_DEV)

        barrier = pltpu.get_barrier_semaphore()
        for nbr in (left, right):
            pl.semaphore_signal(
                barrier, inc=1,
                device_id=(nbr,), device_id_type=pl.DeviceIdType.MESH,
            )
        pl.semaphore_wait(barrier, 2)

        def kv_descs(t):
            g = jnp.mod(my - t, N_DEV)
            descs = []
            for hh in range(HEADS_PER):
                descs.append(pltpu.make_async_copy(
                    k_hbm.at[my, :, g * HEADS_PER + hh, :],
                    kbuf.at[hh], ksem))
                descs.append(pltpu.make_async_copy(
                    v_hbm.at[my, :, g * HEADS_PER + hh, :],
                    vbuf.at[hh], vsem))
            return descs

        comm_wq[0] = wq_ref[...]
        comm_wo[0] = wo_ref[...]

        x2 = x_ref[0]

        row = lax.broadcasted_iota(jnp.int32, (SQ, SQ), 0) // BLK
        col = lax.broadcasted_iota(jnp.int32, (SQ, SQ), 1) // BLK
        maskadd = jnp.where(col <= row, 0.0, -1e9).astype(jnp.float32)

        out_ref[0] = jnp.zeros((SQ, SQ), jnp.float32)

        def step(t, carry):
            cur = lax.rem(t, 2)
            nxt = lax.rem(t + 1, 2)

            descs = kv_descs(t)
            for d in descs:
                d.start()
            for d in descs:
                d.wait()

            q = jnp.dot(x2, comm_wq[cur], preferred_element_type=jnp.float32)
            for hh in range(HEADS_PER):
                qh = q[:, hh * DH:(hh + 1) * DH]
                s = lax.dot_general(
                    qh, kbuf[hh], (((1,), (1,)), ((), ())),
                    preferred_element_type=jnp.float32)
                s = s * SCALE + maskadd
                m = jnp.max(s, axis=-1, keepdims=True)
                w = jnp.exp(s - m)
                w = w * (1.0 / jnp.sum(w, axis=-1, keepdims=True))
                ctx_ref[:, hh * DH:(hh + 1) * DH] = jnp.dot(
                    w, vbuf[hh], preferred_element_type=jnp.float32)
            out_ref[0] += jnp.dot(
                ctx_ref[...], comm_wo[cur], preferred_element_type=jnp.float32)

            @pl.when(t < N_DEV - 1)
            def _():
                send_wq = pltpu.make_async_remote_copy(
                    src_ref=comm_wq.at[cur], dst_ref=comm_wq.at[nxt],
                    send_sem=wq_send.at[cur], recv_sem=wq_recv.at[nxt],
                    device_id=(right,), device_id_type=pl.DeviceIdType.MESH)
                send_wo = pltpu.make_async_remote_copy(
                    src_ref=comm_wo.at[cur], dst_ref=comm_wo.at[nxt],
                    send_sem=wo_send.at[cur], recv_sem=wo_recv.at[nxt],
                    device_id=(right,), device_id_type=pl.DeviceIdType.MESH)
                send_wq.start()
                send_wo.start()
                send_wq.wait()
                send_wo.wait()

            return carry

        lax.fori_loop(0, N_DEV, step, 0)

    return pl.pallas_call(
        body,
        out_shape=jax.ShapeDtypeStruct((1, SQ, SQ), jnp.float32),
        in_specs=[
            pl.BlockSpec(memory_space=pltpu.VMEM),
            pl.BlockSpec(memory_space=pltpu.VMEM),
            pl.BlockSpec(memory_space=pl.ANY),
            pl.BlockSpec(memory_space=pl.ANY),
            pl.BlockSpec(memory_space=pltpu.VMEM),
        ],
        out_specs=pl.BlockSpec(memory_space=pltpu.VMEM),
        scratch_shapes=[
            pltpu.VMEM((2, SQ, SQ), jnp.float32),
            pltpu.VMEM((2, SQ, SQ), jnp.float32),
            pltpu.VMEM((HEADS_PER, SQ, DH), jnp.float32),
            pltpu.VMEM((HEADS_PER, SQ, DH), jnp.float32),
            pltpu.VMEM((SQ, HEADS_PER * DH), jnp.float32),
            pltpu.SemaphoreType.DMA((2,)),
            pltpu.SemaphoreType.DMA((2,)),
            pltpu.SemaphoreType.DMA((2,)),
            pltpu.SemaphoreType.DMA((2,)),
            pltpu.SemaphoreType.DMA,
            pltpu.SemaphoreType.DMA,
        ],
        compiler_params=pltpu.CompilerParams(
            collective_id=0, vmem_limit_bytes=100 * 1024 * 1024),
    )(x, Wq, K_ext, V_ext, Wo)


# device time: 363795 ns/iter; 2.3427x vs baseline; 2.3427x over previous
import jax
import jax.numpy as jnp
from jax import lax
from jax.experimental import pallas as pl
from jax.experimental.pallas import tpu as pltpu

N_DEV = 8
HEADS_PER = 8
DH = 128
SQ = 1024
BLK = 64
SCALE = 0.08838834764831843


def kernel(x, Wq, K_ext, V_ext, Wo):
    def body(x_ref, wq_ref, k_hbm, v_hbm, wo_ref, out_ref,
             comm_wq, comm_wo, kbuf, vbuf, ctx_ref,
             wq_send, wq_recv, wo_send, wo_recv, ksem, vsem):
        my = lax.axis_index("i")
        left = lax.rem(my + N_DEV - 1, N_DEV)
        right = lax.rem(my + 1, N_DEV)

        barrier = pltpu.get_barrier_semaphore()
        for nbr in (left, right):
            pl.semaphore_signal(
                barrier, inc=1,
                device_id=(nbr,), device_id_type=pl.DeviceIdType.MESH,
            )
        pl.semaphore_wait(barrier, 2)

        def kv_descs(t, slot):
            g = jnp.mod(my - t, N_DEV)
            descs = []
            for hh in range(HEADS_PER):
                descs.append(pltpu.make_async_copy(
                    k_hbm.at[my, :, g * HEADS_PER + hh, :],
                    kbuf.at[slot, hh], ksem.at[slot]))
                descs.append(pltpu.make_async_copy(
                    v_hbm.at[my, :, g * HEADS_PER + hh, :],
                    vbuf.at[slot, hh], vsem.at[slot]))
            return descs

        comm_wq[0] = wq_ref[...].astype(jnp.bfloat16)
        comm_wo[0] = wo_ref[...].astype(jnp.bfloat16)
        for d in kv_descs(0, 0):
            d.start()

        x2 = x_ref[0].astype(jnp.bfloat16)

        row = lax.broadcasted_iota(jnp.int32, (SQ, SQ), 0) // BLK
        col = lax.broadcasted_iota(jnp.int32, (SQ, SQ), 1) // BLK
        maskadd = jnp.where(col <= row, 0.0, -1e9).astype(jnp.float32)

        out_ref[0] = jnp.zeros((SQ, SQ), jnp.float32)

        def step(t, carry):
            cur = lax.rem(t, 3)
            nxt = lax.rem(t + 1, 3)
            slot = lax.rem(t, 2)
            nslot = lax.rem(t + 1, 2)

            send_wq = pltpu.make_async_remote_copy(
                src_ref=comm_wq.at[cur], dst_ref=comm_wq.at[nxt],
                send_sem=wq_send.at[cur], recv_sem=wq_recv.at[nxt],
                device_id=(right,), device_id_type=pl.DeviceIdType.MESH)
            send_wo = pltpu.make_async_remote_copy(
                src_ref=comm_wo.at[cur], dst_ref=comm_wo.at[nxt],
                send_sem=wo_send.at[cur], recv_sem=wo_recv.at[nxt],
                device_id=(right,), device_id_type=pl.DeviceIdType.MESH)

            @pl.when(t < N_DEV - 1)
            def _():
                send_wq.start()
                send_wo.start()
                for d in kv_descs(t + 1, nslot):
                    d.start()

            for d in kv_descs(t, slot):
                d.wait()

            q = jnp.dot(x2, comm_wq[cur],
                        preferred_element_type=jnp.float32)
            q = q.astype(jnp.bfloat16)
            for hh in range(HEADS_PER):
                qh = q[:, hh * DH:(hh + 1) * DH]
                kk = kbuf[slot, hh].astype(jnp.bfloat16)
                vv = vbuf[slot, hh].astype(jnp.bfloat16)
                s = lax.dot_general(
                    qh, kk, (((1,), (1,)), ((), ())),
                    preferred_element_type=jnp.float32)
                s = s * SCALE + maskadd
                m = jnp.max(s, axis=-1, keepdims=True)
                w = jnp.exp(s - m)
                w = w * (1.0 / jnp.sum(w, axis=-1, keepdims=True))
                ctx_ref[:, hh * DH:(hh + 1) * DH] = jnp.dot(
                    w.astype(jnp.bfloat16), vv,
                    preferred_element_type=jnp.float32).astype(jnp.bfloat16)
            out_ref[0] += jnp.dot(
                ctx_ref[...], comm_wo[cur],
                preferred_element_type=jnp.float32)

            @pl.when(t < N_DEV - 1)
            def _():
                send_wq.wait()
                send_wo.wait()

            return carry

        lax.fori_loop(0, N_DEV, step, 0)

    return pl.pallas_call(
        body,
        out_shape=jax.ShapeDtypeStruct((1, SQ, SQ), jnp.float32),
        in_specs=[
            pl.BlockSpec(memory_space=pltpu.VMEM),
            pl.BlockSpec(memory_space=pltpu.VMEM),
            pl.BlockSpec(memory_space=pl.ANY),
            pl.BlockSpec(memory_space=pl.ANY),
            pl.BlockSpec(memory_space=pltpu.VMEM),
        ],
        out_specs=pl.BlockSpec(memory_space=pltpu.VMEM),
        scratch_shapes=[
            pltpu.VMEM((3, SQ, SQ), jnp.bfloat16),
            pltpu.VMEM((3, SQ, SQ), jnp.bfloat16),
            pltpu.VMEM((2, HEADS_PER, SQ, DH), jnp.float32),
            pltpu.VMEM((2, HEADS_PER, SQ, DH), jnp.float32),
            pltpu.VMEM((SQ, HEADS_PER * DH), jnp.bfloat16),
            pltpu.SemaphoreType.DMA((3,)),
            pltpu.SemaphoreType.DMA((3,)),
            pltpu.SemaphoreType.DMA((3,)),
            pltpu.SemaphoreType.DMA((3,)),
            pltpu.SemaphoreType.DMA((2,)),
            pltpu.SemaphoreType.DMA((2,)),
        ],
        compiler_params=pltpu.CompilerParams(
            collective_id=0, vmem_limit_bytes=100 * 1024 * 1024),
    )(x, Wq, K_ext, V_ext, Wo)


# device time: 226731 ns/iter; 3.7589x vs baseline; 1.6045x over previous
import jax
import jax.numpy as jnp
from jax import lax
from jax.experimental import pallas as pl
from jax.experimental.pallas import tpu as pltpu

N_DEV = 8
HEADS_PER = 8
DH = 128
SQ = 1024
BLK = 64
SCALE = 0.08838834764831843
R, L = 0, 1


def kernel(x, Wq, K_ext, V_ext, Wo):
    def body(x_ref, wq_ref, k_hbm, v_hbm, wo_ref, out_ref,
             comm, kbuf, vbuf, ctx_ref,
             send_sems, recv_sems, ksem, vsem):
        my = lax.axis_index("i")
        left = lax.rem(my + N_DEV - 1, N_DEV)
        right = lax.rem(my + 1, N_DEV)

        barrier = pltpu.get_barrier_semaphore()
        for nbr in (left, right):
            pl.semaphore_signal(
                barrier, inc=1,
                device_id=(nbr,), device_id_type=pl.DeviceIdType.MESH,
            )
        pl.semaphore_wait(barrier, 2)

        def kv_descs(g):
            descs = []
            for hh in range(HEADS_PER):
                descs.append(pltpu.make_async_copy(
                    k_hbm.at[my, :, g * HEADS_PER + hh, :],
                    kbuf.at[hh], ksem))
                descs.append(pltpu.make_async_copy(
                    v_hbm.at[my, :, g * HEADS_PER + hh, :],
                    vbuf.at[hh], vsem))
            return descs

        def hop(p, dir_):
            tgt = jnp.where(dir_ == R, right, left)
            return pltpu.make_async_remote_copy(
                src_ref=comm.at[dir_, lax.rem(p + 2, 3)],
                dst_ref=comm.at[dir_, lax.rem(p, 3)],
                send_sem=send_sems.at[dir_, lax.rem(p, 3)],
                recv_sem=recv_sems.at[dir_, lax.rem(p, 3)],
                device_id=(tgt,), device_id_type=pl.DeviceIdType.MESH)

        def group_of(j):
            half = (j + 1) // 2
            is_odd = lax.rem(j, 2) == 1
            d = jnp.where(j == 0, 0,
                          jnp.where(j == 7, 4,
                                    jnp.where(is_odd, -half, half)))
            dir_ = jnp.where((j == 0) | (j == 7) | is_odd, R, L)
            p = jnp.where(j == 7, 4, half)
            return jnp.mod(my + d, N_DEV), dir_, p

        wq16 = wq_ref[...].astype(jnp.bfloat16)
        wo16 = wo_ref[...].astype(jnp.bfloat16)
        for dir_ in (R, L):
            comm[dir_, 0, 0] = wq16
            comm[dir_, 0, 1] = wo16
        for dsc in kv_descs(my):
            dsc.start()

        x2 = x_ref[0].astype(jnp.bfloat16)

        rowb = lax.broadcasted_iota(jnp.int32, (SQ, SQ), 0) // BLK
        colb = lax.broadcasted_iota(jnp.int32, (SQ, SQ), 1) // BLK
        maskadd = jnp.where(colb <= rowb, 0.0, -1e9).astype(jnp.float32)

        out_ref[0] = jnp.zeros((SQ, SQ), jnp.float32)

        def step(j, carry):
            g, dir_, p = group_of(j)
            slot = lax.rem(p, 3)
            is_odd = lax.rem(j, 2) == 1

            @pl.when(j == 0)
            def _():
                hop(1, R).start()
                hop(1, L).start()

            @pl.when(j >= 1)
            def _():
                hop(p, dir_).wait()

            @pl.when((is_odd & (j < 7)) | (~is_odd & (j >= 2) & (j <= 4)))
            def _():
                hop(p + 1, dir_).start()

            for dsc in kv_descs(g):
                dsc.wait()

            q = jnp.dot(x2, comm[dir_, slot, 0],
                        preferred_element_type=jnp.float32)
            q = q.astype(jnp.bfloat16)
            for hh in range(HEADS_PER):
                qh = q[:, hh * DH:(hh + 1) * DH]
                kk = kbuf[hh].astype(jnp.bfloat16)
                vv = vbuf[hh].astype(jnp.bfloat16)
                s = lax.dot_general(
                    qh, kk, (((1,), (1,)), ((), ())),
                    preferred_element_type=jnp.float32)
                s = s * SCALE + maskadd
                m = jnp.max(s, axis=-1, keepdims=True)
                w = jnp.exp(s - m)
                w = w * (1.0 / jnp.sum(w, axis=-1, keepdims=True))
                ctx_ref[:, hh * DH:(hh + 1) * DH] = jnp.dot(
                    w.astype(jnp.bfloat16), vv,
                    preferred_element_type=jnp.float32).astype(jnp.bfloat16)
            out_ref[0] += jnp.dot(
                ctx_ref[...], comm[dir_, slot, 1],
                preferred_element_type=jnp.float32)

            @pl.when(j < N_DEV - 1)
            def _():
                gn, _, _ = group_of(j + 1)
                for dsc in kv_descs(gn):
                    dsc.start()

            return carry

        lax.fori_loop(0, N_DEV, step, 0)

    return pl.pallas_call(
        body,
        out_shape=jax.ShapeDtypeStruct((1, SQ, SQ), jnp.float32),
        in_specs=[
            pl.BlockSpec(memory_space=pltpu.VMEM),
            pl.BlockSpec(memory_space=pltpu.VMEM),
            pl.BlockSpec(memory_space=pl.ANY),
            pl.BlockSpec(memory_space=pl.ANY),
            pl.BlockSpec(memory_space=pltpu.VMEM),
        ],
        out_specs=pl.BlockSpec(memory_space=pltpu.VMEM),
        scratch_shapes=[
            pltpu.VMEM((2, 3, 2, SQ, SQ), jnp.bfloat16),
            pltpu.VMEM((HEADS_PER, SQ, DH), jnp.float32),
            pltpu.VMEM((HEADS_PER, SQ, DH), jnp.float32),
            pltpu.VMEM((SQ, HEADS_PER * DH), jnp.bfloat16),
            pltpu.SemaphoreType.DMA((2, 3)),
            pltpu.SemaphoreType.DMA((2, 3)),
            pltpu.SemaphoreType.DMA,
            pltpu.SemaphoreType.DMA,
        ],
        compiler_params=pltpu.CompilerParams(
            collective_id=0, vmem_limit_bytes=100 * 1024 * 1024),
    )(x, Wq, K_ext, V_ext, Wo)


# device time: 216142 ns/iter; 3.9431x vs baseline; 1.0490x over previous
import jax
import jax.numpy as jnp
from jax import lax
from jax.experimental import pallas as pl
from jax.experimental.pallas import tpu as pltpu

N_DEV = 8
HEADS_PER = 8
DH = 128
SQ = 1024
BLK = 64
SCALE = 0.08838834764831843
R, L = 0, 1


def kernel(x, Wq, K_ext, V_ext, Wo):
    def body(x_ref, wq_ref, k_hbm, v_hbm, wo_ref, out_ref,
             comm, kbuf, vbuf, ctx_ref,
             send_sems, recv_sems, ksem, vsem):
        my = lax.axis_index("i")
        left = lax.rem(my + N_DEV - 1, N_DEV)
        right = lax.rem(my + 1, N_DEV)

        barrier = pltpu.get_barrier_semaphore()
        for nbr in (left, right):
            pl.semaphore_signal(
                barrier, inc=1,
                device_id=(nbr,), device_id_type=pl.DeviceIdType.MESH,
            )
        pl.semaphore_wait(barrier, 2)

        def kv_descs(g):
            descs = []
            for hh in range(HEADS_PER):
                descs.append(pltpu.make_async_copy(
                    k_hbm.at[my, :, g * HEADS_PER + hh, :],
                    kbuf.at[hh], ksem))
                descs.append(pltpu.make_async_copy(
                    v_hbm.at[my, :, g * HEADS_PER + hh, :],
                    vbuf.at[hh], vsem))
            return descs

        def hop(p, dir_):
            tgt = jnp.where(dir_ == R, right, left)
            return pltpu.make_async_remote_copy(
                src_ref=comm.at[dir_, lax.rem(p + 2, 3)],
                dst_ref=comm.at[dir_, lax.rem(p, 3)],
                send_sem=send_sems.at[dir_, lax.rem(p, 3)],
                recv_sem=recv_sems.at[dir_, lax.rem(p, 3)],
                device_id=(tgt,), device_id_type=pl.DeviceIdType.MESH)

        def group_of(j):
            half = (j + 1) // 2
            is_odd = lax.rem(j, 2) == 1
            d = jnp.where(j == 0, 0,
                          jnp.where(j == 7, 4,
                                    jnp.where(is_odd, -half, half)))
            dir_ = jnp.where((j == 0) | (j == 7) | is_odd, R, L)
            p = jnp.where(j == 7, 4, half)
            return jnp.mod(my + d, N_DEV), dir_, p

        wq16 = wq_ref[...].astype(jnp.bfloat16)
        wo16 = wo_ref[...].astype(jnp.bfloat16)
        for dir_ in (R, L):
            comm[dir_, 0, 0] = wq16
            comm[dir_, 0, 1] = wo16
        for dsc in kv_descs(my):
            dsc.start()

        x2 = x_ref[0].astype(jnp.bfloat16)

        rowb = lax.broadcasted_iota(jnp.int32, (SQ, SQ), 0) // BLK
        colb = lax.broadcasted_iota(jnp.int32, (SQ, SQ), 1) // BLK
        maskadd = jnp.where(colb <= rowb, 0.0, -1e9).astype(jnp.float32)

        out_ref[0] = jnp.zeros((SQ, SQ), jnp.float32)

        def step(j, carry):
            g, dir_, p = group_of(j)
            slot = lax.rem(p, 3)
            is_odd = lax.rem(j, 2) == 1

            @pl.when(j == 0)
            def _():
                hop(1, R).start()
                hop(1, L).start()

            @pl.when(j >= 1)
            def _():
                hop(p, dir_).wait()

            @pl.when((is_odd & (j < 7)) | (~is_odd & (j >= 2) & (j <= 4)))
            def _():
                hop(p + 1, dir_).start()

            for dsc in kv_descs(g):
                dsc.wait()

            q = jnp.dot(x2, comm[dir_, slot, 0],
                        preferred_element_type=jnp.float32)
            q = q.astype(jnp.bfloat16)
            H = SQ // 2
            for hh in range(HEADS_PER):
                qh = q[:, hh * DH:(hh + 1) * DH]
                kk = kbuf[hh].astype(jnp.bfloat16)
                vv = vbuf[hh].astype(jnp.bfloat16)
                s1 = lax.dot_general(
                    qh[:H], kk[:H], (((1,), (1,)), ((), ())),
                    preferred_element_type=jnp.float32)
                w1 = jnp.exp(s1 * SCALE + maskadd[:H, :H])
                r1 = 1.0 / jnp.sum(w1, axis=-1, keepdims=True)
                c1 = jnp.dot(w1.astype(jnp.bfloat16), vv[:H],
                             preferred_element_type=jnp.float32) * r1
                s2 = lax.dot_general(
                    qh[H:], kk, (((1,), (1,)), ((), ())),
                    preferred_element_type=jnp.float32)
                w2 = jnp.exp(s2 * SCALE + maskadd[H:, :])
                r2 = 1.0 / jnp.sum(w2, axis=-1, keepdims=True)
                c2 = jnp.dot(w2.astype(jnp.bfloat16), vv,
                             preferred_element_type=jnp.float32) * r2
                ctx_ref[:H, hh * DH:(hh + 1) * DH] = c1.astype(jnp.bfloat16)
                ctx_ref[H:, hh * DH:(hh + 1) * DH] = c2.astype(jnp.bfloat16)
            out_ref[0] += jnp.dot(
                ctx_ref[...], comm[dir_, slot, 1],
                preferred_element_type=jnp.float32)

            @pl.when(j < N_DEV - 1)
            def _():
                gn, _, _ = group_of(j + 1)
                for dsc in kv_descs(gn):
                    dsc.start()

            return carry

        lax.fori_loop(0, N_DEV, step, 0)

    return pl.pallas_call(
        body,
        out_shape=jax.ShapeDtypeStruct((1, SQ, SQ), jnp.float32),
        in_specs=[
            pl.BlockSpec(memory_space=pltpu.VMEM),
            pl.BlockSpec(memory_space=pltpu.VMEM),
            pl.BlockSpec(memory_space=pl.ANY),
            pl.BlockSpec(memory_space=pl.ANY),
            pl.BlockSpec(memory_space=pltpu.VMEM),
        ],
        out_specs=pl.BlockSpec(memory_space=pltpu.VMEM),
        scratch_shapes=[
            pltpu.VMEM((2, 3, 2, SQ, SQ), jnp.bfloat16),
            pltpu.VMEM((HEADS_PER, SQ, DH), jnp.float32),
            pltpu.VMEM((HEADS_PER, SQ, DH), jnp.float32),
            pltpu.VMEM((SQ, HEADS_PER * DH), jnp.bfloat16),
            pltpu.SemaphoreType.DMA((2, 3)),
            pltpu.SemaphoreType.DMA((2, 3)),
            pltpu.SemaphoreType.DMA,
            pltpu.SemaphoreType.DMA,
        ],
        compiler_params=pltpu.CompilerParams(
            collective_id=0, vmem_limit_bytes=100 * 1024 * 1024),
    )(x, Wq, K_ext, V_ext, Wo)


# device time: 215417 ns/iter; 3.9564x vs baseline; 1.0034x over previous
import jax
import jax.numpy as jnp
from jax import lax
from jax.experimental import pallas as pl
from jax.experimental.pallas import tpu as pltpu

N_DEV = 8
HEADS_PER = 8
DH = 128
SQ = 1024
BLK = 64
SCALE = 0.08838834764831843
R, L = 0, 1


def kernel(x, Wq, K_ext, V_ext, Wo):
    def body(x_ref, wq_ref, k_hbm, v_hbm, wo_ref, out_ref,
             comm, kbuf, vbuf, ctx_ref,
             send_sems, recv_sems, ksem, vsem):
        my = lax.axis_index("i")
        left = lax.rem(my + N_DEV - 1, N_DEV)
        right = lax.rem(my + 1, N_DEV)

        barrier = pltpu.get_barrier_semaphore()
        for nbr in (left, right):
            pl.semaphore_signal(
                barrier, inc=1,
                device_id=(nbr,), device_id_type=pl.DeviceIdType.MESH,
            )
        pl.semaphore_wait(barrier, 2)

        def kv_descs(g, slot):
            descs = []
            for hh in range(HEADS_PER):
                descs.append(pltpu.make_async_copy(
                    k_hbm.at[my, :, g * HEADS_PER + hh, :],
                    kbuf.at[slot, hh], ksem.at[slot]))
                descs.append(pltpu.make_async_copy(
                    v_hbm.at[my, :, g * HEADS_PER + hh, :],
                    vbuf.at[slot, hh], vsem.at[slot]))
            return descs

        def hop(p, dir_):
            tgt = jnp.where(dir_ == R, right, left)
            return pltpu.make_async_remote_copy(
                src_ref=comm.at[dir_, lax.rem(p + 1, 2)],
                dst_ref=comm.at[dir_, lax.rem(p, 2)],
                send_sem=send_sems.at[dir_, lax.rem(p, 2)],
                recv_sem=recv_sems.at[dir_, lax.rem(p, 2)],
                device_id=(tgt,), device_id_type=pl.DeviceIdType.MESH)

        def group_of(j):
            half = (j + 1) // 2
            is_odd = lax.rem(j, 2) == 1
            d = jnp.where(j == 0, 0,
                          jnp.where(j == 7, 4,
                                    jnp.where(is_odd, -half, half)))
            dir_ = jnp.where((j == 0) | (j == 7) | is_odd, R, L)
            p = jnp.where(j == 7, 4, half)
            return jnp.mod(my + d, N_DEV), dir_, p

        wq16 = wq_ref[...].astype(jnp.bfloat16)
        wo16 = wo_ref[...].astype(jnp.bfloat16)
        for dir_ in (R, L):
            comm[dir_, 0, 0] = wq16
            comm[dir_, 0, 1] = wo16
        for dsc in kv_descs(my, 0):
            dsc.start()

        x2 = x_ref[0].astype(jnp.bfloat16)

        H = SQ // 2
        rowb = lax.broadcasted_iota(jnp.int32, (H, H), 0) // BLK
        colb = lax.broadcasted_iota(jnp.int32, (H, H), 1) // BLK
        mask512 = jnp.where(colb <= rowb, 0.0, -1e9).astype(jnp.float32)

        out_ref[0] = jnp.zeros((SQ, SQ), jnp.float32)

        def step(j, carry):
            g, dir_, p = group_of(j)
            slot = lax.rem(p, 2)
            is_odd = lax.rem(j, 2) == 1

            @pl.when(j < N_DEV - 1)
            def _():
                gn, _, _ = group_of(j + 1)
                for dsc in kv_descs(gn, lax.rem(j + 1, 2)):
                    dsc.start()

            @pl.when(j == 0)
            def _():
                hop(1, R).start()
                hop(1, L).start()

            @pl.when(j >= 1)
            def _():
                hop(p, dir_).wait()

            @pl.when((is_odd & (j < 7)) | (~is_odd & (j >= 2) & (j <= 4)))
            def _():
                hop(p + 1, dir_).start()

            kvslot = lax.rem(j, 2)
            for dsc in kv_descs(g, kvslot):
                dsc.wait()

            q = jnp.dot(x2, comm[dir_, slot, 0],
                        preferred_element_type=jnp.float32)
            q = (q * SCALE).astype(jnp.bfloat16)
            for hh in range(HEADS_PER):
                qh = q[:, hh * DH:(hh + 1) * DH]
                kk = kbuf[kvslot, hh].astype(jnp.bfloat16)
                vv = vbuf[kvslot, hh].astype(jnp.bfloat16)
                s1 = lax.dot_general(
                    qh[:H], kk[:H], (((1,), (1,)), ((), ())),
                    preferred_element_type=jnp.float32)
                w1 = jnp.exp(s1 + mask512)
                r1 = 1.0 / jnp.sum(w1, axis=-1, keepdims=True)
                c1 = jnp.dot(w1.astype(jnp.bfloat16), vv[:H],
                             preferred_element_type=jnp.float32) * r1
                s2a = lax.dot_general(
                    qh[H:], kk[:H], (((1,), (1,)), ((), ())),
                    preferred_element_type=jnp.float32)
                s2b = lax.dot_general(
                    qh[H:], kk[H:], (((1,), (1,)), ((), ())),
                    preferred_element_type=jnp.float32)
                w2a = jnp.exp(s2a)
                w2b = jnp.exp(s2b + mask512)
                r2 = 1.0 / (jnp.sum(w2a, axis=-1, keepdims=True)
                            + jnp.sum(w2b, axis=-1, keepdims=True))
                c2 = (jnp.dot(w2a.astype(jnp.bfloat16), vv[:H],
                              preferred_element_type=jnp.float32)
                      + jnp.dot(w2b.astype(jnp.bfloat16), vv[H:],
                                preferred_element_type=jnp.float32)) * r2
                ctx_ref[:H, hh * DH:(hh + 1) * DH] = c1.astype(jnp.bfloat16)
                ctx_ref[H:, hh * DH:(hh + 1) * DH] = c2.astype(jnp.bfloat16)
            out_ref[0] += jnp.dot(
                ctx_ref[...], comm[dir_, slot, 1],
                preferred_element_type=jnp.float32)

            return carry

        lax.fori_loop(0, N_DEV, step, 0)

    return pl.pallas_call(
        body,
        out_shape=jax.ShapeDtypeStruct((1, SQ, SQ), jnp.float32),
        in_specs=[
            pl.BlockSpec(memory_space=pltpu.VMEM),
            pl.BlockSpec(memory_space=pltpu.VMEM),
            pl.BlockSpec(memory_space=pl.ANY),
            pl.BlockSpec(memory_space=pl.ANY),
            pl.BlockSpec(memory_space=pltpu.VMEM),
        ],
        out_specs=pl.BlockSpec(memory_space=pltpu.VMEM),
        scratch_shapes=[
            pltpu.VMEM((2, 2, 2, SQ, SQ), jnp.bfloat16),
            pltpu.VMEM((2, HEADS_PER, SQ, DH), jnp.float32),
            pltpu.VMEM((2, HEADS_PER, SQ, DH), jnp.float32),
            pltpu.VMEM((SQ, HEADS_PER * DH), jnp.bfloat16),
            pltpu.SemaphoreType.DMA((2, 2)),
            pltpu.SemaphoreType.DMA((2, 2)),
            pltpu.SemaphoreType.DMA((2,)),
            pltpu.SemaphoreType.DMA((2,)),
        ],
        compiler_params=pltpu.CompilerParams(
            collective_id=0, vmem_limit_bytes=100 * 1024 * 1024),
    )(x, Wq, K_ext, V_ext, Wo)


# device time: 210837 ns/iter; 4.0423x vs baseline; 1.0217x over previous
import jax
import jax.numpy as jnp
from jax import lax
from jax.experimental import pallas as pl
from jax.experimental.pallas import tpu as pltpu

N_DEV = 8
HEADS_PER = 8
DH = 128
SQ = 1024
BLK = 64
SCALE = 0.08838834764831843
R, L = 0, 1


def kernel(x, Wq, K_ext, V_ext, Wo):
    def body(x_ref, wq_ref, k_hbm, v_hbm, wo_ref, out_ref,
             comm, kbuf, vbuf, ctx_ref,
             send_sems, recv_sems, ksem, vsem):
        my = lax.axis_index("i")
        left = lax.rem(my + N_DEV - 1, N_DEV)
        right = lax.rem(my + 1, N_DEV)

        barrier = pltpu.get_barrier_semaphore()
        for nbr in (left, right):
            pl.semaphore_signal(
                barrier, inc=1,
                device_id=(nbr,), device_id_type=pl.DeviceIdType.MESH,
            )
        pl.semaphore_wait(barrier, 2)

        def kv_descs(g, slot):
            descs = []
            for hh in range(HEADS_PER):
                descs.append(pltpu.make_async_copy(
                    k_hbm.at[my, :, g * HEADS_PER + hh, :],
                    kbuf.at[slot, hh], ksem.at[slot]))
                descs.append(pltpu.make_async_copy(
                    v_hbm.at[my, :, g * HEADS_PER + hh, :],
                    vbuf.at[slot, hh], vsem.at[slot]))
            return descs

        def hop(p, dir_, half):
            tgt = jnp.where(dir_ == R, right, left)
            hs = half * (SQ // 2)
            return pltpu.make_async_remote_copy(
                src_ref=comm.at[dir_, lax.rem(p + 1, 2), :,
                                pl.ds(hs, SQ // 2), :],
                dst_ref=comm.at[dir_, lax.rem(p, 2), :,
                                pl.ds(hs, SQ // 2), :],
                send_sem=send_sems.at[dir_, lax.rem(p, 2), half],
                recv_sem=recv_sems.at[dir_, lax.rem(p, 2), half],
                device_id=(tgt,), device_id_type=pl.DeviceIdType.MESH)

        def group_of(j):
            half = (j + 1) // 2
            is_odd = lax.rem(j, 2) == 1
            d = jnp.where(j == 0, 0,
                          jnp.where(j == 7, 4,
                                    jnp.where(is_odd, -half, half)))
            dir_ = jnp.where((j == 0) | (j == 7) | is_odd, R, L)
            p = jnp.where(j == 7, 4, half)
            return jnp.mod(my + d, N_DEV), dir_, p

        wq16 = wq_ref[...].astype(jnp.bfloat16)
        wo16 = wo_ref[...].astype(jnp.bfloat16)
        for dir_ in (R, L):
            comm[dir_, 0, 0] = wq16
            comm[dir_, 0, 1] = wo16
        for dsc in kv_descs(my, 0):
            dsc.start()

        x2 = x_ref[0].astype(jnp.bfloat16)

        H = SQ // 2
        rowb = lax.broadcasted_iota(jnp.int32, (H, H), 0) // BLK
        colb = lax.broadcasted_iota(jnp.int32, (H, H), 1) // BLK
        mask512 = jnp.where(colb <= rowb, 0.0, -1e9).astype(jnp.float32)

        out_ref[0] = jnp.zeros((SQ, SQ), jnp.float32)

        def step(j, carry):
            g, dir_, p = group_of(j)
            slot = lax.rem(p, 2)
            is_odd = lax.rem(j, 2) == 1

            @pl.when(j < N_DEV - 1)
            def _():
                gn, _, _ = group_of(j + 1)
                for dsc in kv_descs(gn, lax.rem(j + 1, 2)):
                    dsc.start()

            @pl.when(j == 0)
            def _():
                for half in (0, 1):
                    hop(1, R, half).start()
                    hop(1, L, half).start()

            fwd = (is_odd & (j < 7)) | (~is_odd & (j >= 2) & (j <= 4))
            for half in (0, 1):
                @pl.when(j >= 1)
                def _(half=half):
                    hop(p, dir_, half).wait()

                @pl.when(fwd)
                def _(half=half):
                    hop(p + 1, dir_, half).start()

            kvslot = lax.rem(j, 2)
            for dsc in kv_descs(g, kvslot):
                dsc.wait()

            q = jnp.dot(x2, comm[dir_, slot, 0],
                        preferred_element_type=jnp.float32)
            q = (q * SCALE).astype(jnp.bfloat16)
            for hh in range(HEADS_PER):
                qh = q[:, hh * DH:(hh + 1) * DH]
                kk = kbuf[kvslot, hh].astype(jnp.bfloat16)
                vv = vbuf[kvslot, hh].astype(jnp.bfloat16)
                s1 = lax.dot_general(
                    qh[:H], kk[:H], (((1,), (1,)), ((), ())),
                    preferred_element_type=jnp.float32)
                w1 = jnp.exp(s1 + mask512)
                r1 = 1.0 / jnp.sum(w1, axis=-1, keepdims=True)
                c1 = jnp.dot(w1.astype(jnp.bfloat16), vv[:H],
                             preferred_element_type=jnp.float32) * r1
                s2a = lax.dot_general(
                    qh[H:], kk[:H], (((1,), (1,)), ((), ())),
                    preferred_element_type=jnp.float32)
                s2b = lax.dot_general(
                    qh[H:], kk[H:], (((1,), (1,)), ((), ())),
                    preferred_element_type=jnp.float32)
                w2a = jnp.exp(s2a)
                w2b = jnp.exp(s2b + mask512)
                r2 = 1.0 / (jnp.sum(w2a, axis=-1, keepdims=True)
                            + jnp.sum(w2b, axis=-1, keepdims=True))
                c2 = (jnp.dot(w2a.astype(jnp.bfloat16), vv[:H],
                              preferred_element_type=jnp.float32)
                      + jnp.dot(w2b.astype(jnp.bfloat16), vv[H:],
                                preferred_element_type=jnp.float32)) * r2
                ctx_ref[:H, hh * DH:(hh + 1) * DH] = c1.astype(jnp.bfloat16)
                ctx_ref[H:, hh * DH:(hh + 1) * DH] = c2.astype(jnp.bfloat16)
            out_ref[0] += jnp.dot(
                ctx_ref[...], comm[dir_, slot, 1],
                preferred_element_type=jnp.float32)

            return carry

        lax.fori_loop(0, N_DEV, step, 0)

    return pl.pallas_call(
        body,
        out_shape=jax.ShapeDtypeStruct((1, SQ, SQ), jnp.float32),
        in_specs=[
            pl.BlockSpec(memory_space=pltpu.VMEM),
            pl.BlockSpec(memory_space=pltpu.VMEM),
            pl.BlockSpec(memory_space=pl.ANY),
            pl.BlockSpec(memory_space=pl.ANY),
            pl.BlockSpec(memory_space=pltpu.VMEM),
        ],
        out_specs=pl.BlockSpec(memory_space=pltpu.VMEM),
        scratch_shapes=[
            pltpu.VMEM((2, 2, 2, SQ, SQ), jnp.bfloat16),
            pltpu.VMEM((2, HEADS_PER, SQ, DH), jnp.float32),
            pltpu.VMEM((2, HEADS_PER, SQ, DH), jnp.float32),
            pltpu.VMEM((SQ, HEADS_PER * DH), jnp.bfloat16),
            pltpu.SemaphoreType.DMA((2, 2, 2)),
            pltpu.SemaphoreType.DMA((2, 2, 2)),
            pltpu.SemaphoreType.DMA((2,)),
            pltpu.SemaphoreType.DMA((2,)),
        ],
        compiler_params=pltpu.CompilerParams(
            collective_id=0, vmem_limit_bytes=100 * 1024 * 1024),
    )(x, Wq, K_ext, V_ext, Wo)
